# Initial kernel scaffold; baseline (speedup 1.0000x reference)
import jax
import jax.numpy as jnp
from jax.experimental import pallas as pl


def kernel(z, table):
    raise NotImplementedError("write your pallas kernel here")


# speedup vs baseline: 1.4736x; 1.4736x over previous
import functools

import jax
import jax.numpy as jnp
from jax import lax
from jax.experimental import pallas as pl
from jax.experimental.pallas import tpu as pltpu
from jax.experimental.pallas import tpu_sc as plsc

_NODE_NF = 1000000
_HIDDEN = 32
_BATCH = 16384
_FIELDS = 26

_B = _BATCH * _FIELDS
_NC = 2
_NS = 16
_NW = _NC * _NS
_CHUNK = 128
_ROWS_PER_W = _B // _NW
_CHUNKS_PER_W = _ROWS_PER_W // _CHUNK

_mesh = plsc.VectorSubcoreMesh(core_axis_name="c", subcore_axis_name="s")


@functools.partial(
    pl.kernel,
    mesh=_mesh,
    compiler_params=pltpu.CompilerParams(use_tc_tiling_on_sc=False),
    out_type=jax.ShapeDtypeStruct((_B, _HIDDEN), jnp.float32),
    scratch_types=[
        pltpu.VMEM((_CHUNKS_PER_W, _CHUNK), jnp.int32),
        pltpu.VMEM((2, _CHUNK, _HIDDEN), jnp.float32),
        pltpu.SemaphoreType.DMA,
    ],
)
def _sc_gather(z_hbm, table_hbm, out_hbm, idx_v, rows_v, gsem):
    wid = lax.axis_index("s") * _NC + lax.axis_index("c")
    row_base = wid * _ROWS_PER_W
    chunk_base = wid * _CHUNKS_PER_W

    pltpu.sync_copy(z_hbm.at[pl.ds(chunk_base, _CHUNKS_PER_W)], idx_v)

    pltpu.async_copy(table_hbm.at[idx_v.at[0]], rows_v.at[0], gsem)

    def body(c, carry):
        cur = lax.rem(c, 2)
        nxt = lax.rem(c + 1, 2)
        pltpu.make_async_copy(
            table_hbm.at[idx_v.at[c]], rows_v.at[cur], gsem
        ).wait()
        pltpu.async_copy(table_hbm.at[idx_v.at[c + 1]], rows_v.at[nxt], gsem)
        pltpu.sync_copy(
            rows_v.at[cur],
            out_hbm.at[pl.ds(row_base + c * _CHUNK, _CHUNK)],
        )
        return carry

    lax.fori_loop(0, _CHUNKS_PER_W - 1, body, 0)

    last = _CHUNKS_PER_W - 1
    pltpu.make_async_copy(
        table_hbm.at[idx_v.at[last]], rows_v.at[last % 2], gsem
    ).wait()
    pltpu.sync_copy(
        rows_v.at[last % 2],
        out_hbm.at[pl.ds(row_base + last * _CHUNK, _CHUNK)],
    )


def kernel(z, table):
    zf = z.reshape(_NW * _CHUNKS_PER_W, _CHUNK)
    out = _sc_gather(zf, table)
    return (out.reshape(_BATCH, _FIELDS, _HIDDEN), 0)


# speedup vs baseline: 1.5693x; 1.0649x over previous
import functools

import jax
import jax.numpy as jnp
from jax import lax
from jax.experimental import pallas as pl
from jax.experimental.pallas import tpu as pltpu
from jax.experimental.pallas import tpu_sc as plsc

_NODE_NF = 1000000
_HIDDEN = 32
_BATCH = 16384
_FIELDS = 26

_B = _BATCH * _FIELDS
_NC = 2
_NS = 16
_NW = _NC * _NS
_CHUNK = 128
_K = 13
_GROUPS = 8
_ROWS_PER_W = _CHUNK * _K * _GROUPS
_CHUNKS_PER_W = _K * _GROUPS

_mesh = plsc.VectorSubcoreMesh(core_axis_name="c", subcore_axis_name="s")


@functools.partial(
    pl.kernel,
    mesh=_mesh,
    compiler_params=pltpu.CompilerParams(use_tc_tiling_on_sc=False),
    out_type=jax.ShapeDtypeStruct((_B, _HIDDEN), jnp.float32),
    scratch_types=[
        pltpu.VMEM((_CHUNKS_PER_W, _CHUNK), jnp.int32),
        pltpu.VMEM((_K, _CHUNK, _HIDDEN), jnp.float32),
        pltpu.VMEM((_K, _CHUNK, _HIDDEN), jnp.float32),
        pltpu.SemaphoreType.DMA,
        pltpu.SemaphoreType.DMA,
        pltpu.SemaphoreType.DMA,
        pltpu.SemaphoreType.DMA,
    ],
)
def _sc_gather(z_hbm, table_hbm, out_hbm, idx_v, bufa, bufb, gsa, gsb, osa, osb):
    wid = lax.axis_index("s") * _NC + lax.axis_index("c")
    row_base = wid * _ROWS_PER_W
    chunk_base = wid * _CHUNKS_PER_W

    pltpu.sync_copy(z_hbm.at[pl.ds(chunk_base, _CHUNKS_PER_W)], idx_v)

    def fire_gathers(g, buf, sem):
        for j in range(_K):
            pltpu.async_copy(table_hbm.at[idx_v.at[g * _K + j]], buf.at[j], sem)

    def drain(sem, buf):
        for j in range(_K):
            pltpu.make_async_copy(table_hbm.at[idx_v.at[0]], buf.at[j], sem).wait()

    def fire_writes(g, buf, sem):
        for j in range(_K):
            pltpu.async_copy(
                buf.at[j],
                out_hbm.at[pl.ds(row_base + (g * _K + j) * _CHUNK, _CHUNK)],
                sem,
            )

    fire_gathers(0, bufa, gsa)
    fire_gathers(1, bufb, gsb)

    def body(i, carry):
        ga = 2 * i
        drain(gsa, bufa)
        fire_writes(ga, bufa, osa)
        drain(gsb, bufb)
        fire_writes(ga + 1, bufb, osb)
        drain(osa, bufa)
        fire_gathers(ga + 2, bufa, gsa)
        drain(osb, bufb)
        fire_gathers(ga + 3, bufb, gsb)
        return carry

    lax.fori_loop(0, _GROUPS // 2 - 1, body, 0)

    last = _GROUPS - 2
    drain(gsa, bufa)
    fire_writes(last, bufa, osa)
    drain(gsb, bufb)
    fire_writes(last + 1, bufb, osb)
    drain(osa, bufa)
    drain(osb, bufb)


def kernel(z, table):
    zf = z.reshape(_NW * _CHUNKS_PER_W, _CHUNK)
    out = _sc_gather(zf, table)
    return (out.reshape(_BATCH, _FIELDS, _HIDDEN), 0)
